# baseline (speedup 1.0000x reference)
import functools

import jax
import jax.numpy as jnp
from jax import lax
from jax.experimental import pallas as pl
from jax.experimental.pallas import tpu as pltpu
from jax.experimental.pallas import tpu_sc as plsc

NC = 2
NS = 16
LANES = 16

EDGE_CHUNK = 80


def _matmul(x, W):
    n, d_in = x.shape
    d_out = W.shape[1]
    blk = 1000

    def body(x_ref, w_ref, o_ref):
        o_ref[...] = jnp.dot(x_ref[...], w_ref[...],
                             preferred_element_type=jnp.float32)

    return pl.pallas_call(
        body,
        grid=(n // blk,),
        in_specs=[
            pl.BlockSpec((blk, d_in), lambda i: (i, 0)),
            pl.BlockSpec((d_in, d_out), lambda i: (0, 0)),
        ],
        out_specs=pl.BlockSpec((blk, d_out), lambda i: (i, 0)),
        out_shape=jax.ShapeDtypeStruct((n, d_out), jnp.float32),
    )(x, W)


def _combine_relu(partials, n):
    _, _, d = partials.shape
    blk = 1000

    def body(p_ref, o_ref):
        o_ref[...] = jnp.maximum(p_ref[0] + p_ref[1], 0.0)

    return pl.pallas_call(
        body,
        grid=(n // blk,),
        in_specs=[pl.BlockSpec((2, blk, d), lambda i: (0, i, 0))],
        out_specs=pl.BlockSpec((blk, d), lambda i: (i, 0)),
        out_shape=jax.ShapeDtypeStruct((n, d), jnp.float32),
    )(partials)


def _sc_edge_aggregate(xw, row, col, vals):
    n, d = xw.shape
    e = row.shape[0]
    ntiles = NC * NS
    e_per_tile = e // ntiles
    k = EDGE_CHUNK
    nch = e_per_tile // k
    zrows = 8
    n_pad = ((n + NS * zrows - 1) // (NS * zrows)) * (NS * zrows)
    rows_per_tile = n_pad // NS
    nzch = rows_per_tile // zrows

    nbuf = 4

    mesh = plsc.VectorSubcoreMesh(core_axis_name="c", subcore_axis_name="s")

    @functools.partial(
        pl.kernel,
        mesh=mesh,
        out_type=jax.ShapeDtypeStruct((NC, n_pad, d), jnp.float32),
        scratch_types=(
            [pltpu.VMEM((k,), jnp.int32) for _ in range(nbuf)]
            + [pltpu.VMEM((k,), jnp.int32) for _ in range(nbuf)]
            + [pltpu.VMEM((k,), jnp.float32) for _ in range(nbuf)]
            + [pltpu.VMEM((k, d), jnp.float32) for _ in range(nbuf)]
            + [
                pltpu.VMEM((zrows, d), jnp.float32),
                pltpu.VMEM_SHARED((n_pad, d), jnp.float32),
                pltpu.SemaphoreType.DMA,
                pltpu.SemaphoreType.DMA,
                pltpu.SemaphoreType.DMA,
                pltpu.SemaphoreType.DMA,
                pltpu.SemaphoreType.DMA,
                pltpu.SemaphoreType.DMA,
            ]
        ),
    )
    def sc_kernel(xw_hbm, row_hbm, col_hbm, val_hbm, out_hbm, *refs):
        colvs = refs[0:nbuf]
        rowvs = refs[nbuf:2 * nbuf]
        valvs = refs[2 * nbuf:3 * nbuf]
        bufs = refs[3 * nbuf:4 * nbuf]
        zb, acc, zsem, isem, rsem, vsem, gsem, ssem = refs[4 * nbuf:]
        c = lax.axis_index("c")
        s = lax.axis_index("s")
        tile = c * NS + s
        ebase = tile * e_per_tile

        def zfill(i, _):
            for dd in range(d // LANES):
                zb[i, pl.ds(dd * LANES, LANES)] = jnp.zeros((LANES,),
                                                            jnp.float32)
            return 0
        lax.fori_loop(0, zrows, zfill, 0)
        rbase = s * rows_per_tile

        def zcopy(i, _):
            pltpu.async_copy(zb, acc.at[pl.ds(rbase + i * zrows, zrows)],
                             zsem)
            return 0
        lax.fori_loop(0, nzch, zcopy, 0)

        def zwait(i, _):
            pltpu.make_async_copy(zb, acc.at[pl.ds(rbase, zrows)],
                                  zsem).wait()
            return 0

        def start_col(i, colv):
            pltpu.async_copy(col_hbm.at[pl.ds(ebase + i * k, k)], colv,
                             isem)

        def wait_col(i, colv):
            pltpu.make_async_copy(col_hbm.at[pl.ds(ebase + i * k, k)],
                                  colv, isem).wait()

        def start_row(i, rowv):
            pltpu.async_copy(row_hbm.at[pl.ds(ebase + i * k, k)], rowv,
                             rsem)

        def wait_row(i, rowv):
            pltpu.make_async_copy(row_hbm.at[pl.ds(ebase + i * k, k)],
                                  rowv, rsem).wait()

        def start_val(i, valv):
            pltpu.async_copy(val_hbm.at[pl.ds(ebase + i * k, k)], valv,
                             vsem)

        def wait_val(i, valv):
            pltpu.make_async_copy(val_hbm.at[pl.ds(ebase + i * k, k)],
                                  valv, vsem).wait()

        kh = k // 2

        def start_gather(i, buf, colv):
            pltpu.async_copy(xw_hbm.at[colv.at[pl.ds(0, kh)]],
                             buf.at[pl.ds(0, kh)], gsem)
            pltpu.async_copy(xw_hbm.at[colv.at[pl.ds(kh, kh)]],
                             buf.at[pl.ds(kh, kh)], gsem)

        def wait_gather(i, buf, colv):
            pltpu.make_async_copy(xw_hbm.at[colv.at[pl.ds(0, kh)]],
                                  buf.at[pl.ds(0, kh)], gsem).wait()
            pltpu.make_async_copy(xw_hbm.at[colv.at[pl.ds(kh, kh)]],
                                  buf.at[pl.ds(kh, kh)], gsem).wait()

        def start_scatter(i, buf, rowv):
            pltpu.async_copy(buf, acc.at[rowv], ssem, add=True)

        def wait_scatter(i, buf, rowv):
            pltpu.make_async_copy(buf, acc.at[rowv], ssem).wait()

        def scale(i, buf, valv):
            def group(g, _):
                vv16 = valv[pl.ds(g * LANES, LANES)]
                for jj in range(LANES):
                    bc = jnp.full((LANES,), vv16[jj], jnp.float32)
                    j = g * LANES + jj
                    for dd in range(d // LANES):
                        sl = pl.ds(dd * LANES, LANES)
                        buf[j, sl] = buf[j, sl] * bc
                return 0
            lax.fori_loop(0, k // LANES, group, 0)

        def prime(i, r):
            wait_col(i, colvs[r])
            start_gather(i, bufs[r], colvs[r])
            start_row(i, rowvs[r])
            start_val(i, valvs[r])

        def chunk(i, r, scat_wait=True, issue=True, col_issue=True):
            wait_gather(i, bufs[r], colvs[r])
            if scat_wait:
                rp = (r + nbuf - 2) % nbuf
                wait_scatter(i - 2, bufs[rp], rowvs[rp])
            if issue:
                prime(i + 2, (r + 2) % nbuf)
            if col_issue:
                start_col(i + 3, colvs[(r + 3) % nbuf])
            wait_val(i, valvs[r])
            wait_row(i, rowvs[r])
            scale(i, bufs[r], valvs[r])
            start_scatter(i, bufs[r], rowvs[r])

        nloop = (nch - 5) // nbuf
        assert 2 + nbuf * nloop == nch - 3 and nch >= 9
        for i in range(3):
            start_col(i, colvs[i])
        prime(0, 0)
        prime(1, 1)
        lax.fori_loop(0, nzch, zwait, 0)
        plsc.subcore_barrier()
        chunk(0, 0, scat_wait=False)
        chunk(1, 1, scat_wait=False)

        def loop_body(t, _):
            base = 2 + nbuf * t
            for slot in range(nbuf):
                chunk(base + slot, (2 + slot) % nbuf)
            return 0
        lax.fori_loop(0, nloop, loop_body, 0)

        for i in (nch - 3, nch - 2, nch - 1):
            chunk(i, i % nbuf, issue=(i + 2 <= nch - 1),
                  col_issue=(i + 3 <= nch - 1))
        for i in (nch - 2, nch - 1):
            rl = i % nbuf
            wait_scatter(i, bufs[rl], rowvs[rl])

        plsc.subcore_barrier()
        pltpu.sync_copy(acc.at[pl.ds(rbase, rows_per_tile)],
                        out_hbm.at[c, pl.ds(rbase, rows_per_tile)])

    return sc_kernel(xw, row, col, vals)


def kernel(x, adj_indices, adj_values, W):
    row = adj_indices[0].astype(jnp.int32)
    col = adj_indices[1].astype(jnp.int32)
    xw = _matmul(x, W)
    partials = _sc_edge_aggregate(xw, row, col, adj_values)
    return _combine_relu(partials, x.shape[0])
